# baseline (speedup 1.0000x reference)
import functools

import jax
import jax.numpy as jnp
from jax import lax
from jax.experimental import pallas as pl
from jax.experimental.pallas import tpu as pltpu
from jax.experimental.pallas import tpu_sc as plsc

_RES = 16384
_LEAK = 0.6
_BIAS = 1.6
_NC = 2
_NS = 16
_NW = _NC * _NS
_CHUNK = 4096
_NBUF = 4
_R0, _R1 = 21, 21


def _sc_partials(rows, cols, vals, res_state):
    nnz = rows.shape[0]
    nnz16 = (nnz // 16) * 16
    n_chunks_total = nnz16 // _CHUNK
    rem = nnz16 - n_chunks_total * _CHUNK
    assert nnz16 >= _CHUNK

    c0_total = (n_chunks_total * _R0) // (_R0 + _R1)
    c1_total = n_chunks_total - c0_total
    q0, r0 = divmod(c0_total, _NS)
    q1, r1 = divmod(c1_total, _NS)
    assert min(q0, q1) >= _NBUF

    mesh = plsc.VectorSubcoreMesh(core_axis_name="c", subcore_axis_name="s")

    @functools.partial(
        pl.kernel,
        out_type=jax.ShapeDtypeStruct((2 * _NW, _RES), jnp.float32),
        mesh=mesh,
        scratch_types=[
            pltpu.VMEM((_RES,), jnp.float32),
            pltpu.VMEM((_RES,), jnp.float32),
            pltpu.VMEM((_RES,), jnp.float32),
            pltpu.VMEM((_NBUF, _CHUNK), jnp.int32),
            pltpu.VMEM((_NBUF, _CHUNK), jnp.int32),
            pltpu.VMEM((_NBUF, _CHUNK), jnp.float32),
            pltpu.VMEM((max(rem, 16),), jnp.int32),
            pltpu.VMEM((max(rem, 16),), jnp.int32),
            pltpu.VMEM((max(rem, 16),), jnp.float32),
            pltpu.SemaphoreType.DMA((_NBUF,)),
            pltpu.SemaphoreType.DMA,
        ],
        compiler_params=pltpu.CompilerParams(needs_layout_passes=False),
    )
    def body(rows_h, cols_h, vals_h, res_h, out_h,
             res_v, acc_v, acc_b, rbuf, cbuf, vbuf, rrem_v, crem_v, vrem_v,
             sems, res_sem):
        cid = lax.axis_index("c")
        sid = lax.axis_index("s")
        wid = sid * _NC + cid
        is_c0 = cid == 0
        is_mopup = is_c0 & (sid == _NS - 1)
        n_chunks = jnp.where(is_c0, q0 + (sid < r0), q1 + (sid < r1))
        start_chunk = jnp.where(
            is_c0,
            sid * q0 + jnp.minimum(sid, r0),
            c0_total + sid * q1 + jnp.minimum(sid, r1),
        )
        base = start_chunk * _CHUNK

        def issue(ci, slot):
            off = base + ci * _CHUNK
            pltpu.async_copy(rows_h.at[pl.ds(off, _CHUNK)], rbuf.at[slot], sems.at[slot])
            pltpu.async_copy(cols_h.at[pl.ds(off, _CHUNK)], cbuf.at[slot], sems.at[slot])
            pltpu.async_copy(vals_h.at[pl.ds(off, _CHUNK)], vbuf.at[slot], sems.at[slot])

        def drain(slot):
            pltpu.make_async_copy(rows_h.at[pl.ds(0, _CHUNK)], rbuf.at[slot], sems.at[slot]).wait()
            pltpu.make_async_copy(cols_h.at[pl.ds(0, _CHUNK)], cbuf.at[slot], sems.at[slot]).wait()
            pltpu.make_async_copy(vals_h.at[pl.ds(0, _CHUNK)], vbuf.at[slot], sems.at[slot]).wait()

        def gather_scatter(acc, slot, j):
            sl = pl.ds(j * 16, 16)
            g = plsc.load_gather(res_v, [cbuf[slot, sl]])
            plsc.addupdate_scatter(acc, [rbuf[slot, sl]], vbuf[slot, sl] * g)

        for b in range(_NBUF):
            issue(b, b)
        res_copy = pltpu.async_copy(res_h, res_v, res_sem)

        zeros = jnp.zeros((16,), jnp.float32)

        @plsc.parallel_loop(0, _RES // 16, unroll=8)
        def _(i):
            acc_v[pl.ds(i * 16, 16)] = zeros
            acc_b[pl.ds(i * 16, 16)] = zeros

        res_copy.wait()

        def cbody(ci, _):
            slot = lax.rem(ci, _NBUF)
            drain(slot)

            @plsc.parallel_loop(0, _CHUNK // 16, step=2, unroll=4)
            def _(j):
                gather_scatter(acc_v, slot, j)
                gather_scatter(acc_b, slot, j + 1)

            @pl.when(ci + _NBUF < n_chunks)
            def _():
                issue(ci + _NBUF, slot)

            return ()

        lax.fori_loop(0, n_chunks, cbody, ())

        if rem > 0:
            roff = pl.multiple_of(jnp.where(sid >= 0, nnz16 - rem, 0), _CHUNK)
            pltpu.async_copy(rows_h.at[pl.ds(roff, rem)], rrem_v, res_sem).wait()
            pltpu.async_copy(cols_h.at[pl.ds(roff, rem)], crem_v, res_sem).wait()
            pltpu.async_copy(vals_h.at[pl.ds(roff, rem)], vrem_v, res_sem).wait()

            @pl.when(is_mopup)
            def _():
                @plsc.parallel_loop(0, rem // 16, unroll=4)
                def _(j):
                    sl = pl.ds(j * 16, 16)
                    g = plsc.load_gather(res_v, [crem_v[sl]])
                    plsc.addupdate_scatter(acc_v, [rrem_v[sl]], vrem_v[sl] * g)

        pltpu.sync_copy(acc_v, out_h.at[wid])
        pltpu.sync_copy(acc_b, out_h.at[_NW + wid])

    return body(rows, cols, vals, res_state)


def _reduce_epilogue(partials, proj_vars, res_state, tail_rows, tail_contrib):
    n_tail = tail_rows.shape[0]

    def body(tr_ref, tc_ref, p_ref, pv_ref, rs_ref, o_ref):
        s = jnp.sum(p_ref[...], axis=0)
        iota = lax.broadcasted_iota(jnp.int32, (_RES,), 0)
        for i in range(n_tail):
            s = s + jnp.where(iota == tr_ref[i], tc_ref[i], 0.0)
        act = jnp.tanh(s + pv_ref[...] + _BIAS)
        o_ref[...] = _LEAK * act + (1.0 - _LEAK) * rs_ref[...]

    return pl.pallas_call(
        body,
        out_shape=jax.ShapeDtypeStruct((_RES,), jnp.float32),
        in_specs=[
            pl.BlockSpec(memory_space=pltpu.SMEM),
            pl.BlockSpec(memory_space=pltpu.SMEM),
            pl.BlockSpec(memory_space=pltpu.VMEM),
            pl.BlockSpec(memory_space=pltpu.VMEM),
            pl.BlockSpec(memory_space=pltpu.VMEM),
        ],
    )(tail_rows, tail_contrib, partials, proj_vars, res_state)


def kernel(proj_vars, res_state, rows, cols, vals):
    nnz = rows.shape[0]
    nnz16 = (nnz // 16) * 16
    tail = nnz - nnz16
    rows32 = rows.astype(jnp.int32)
    cols32 = cols.astype(jnp.int32)
    vals32 = vals.astype(jnp.float32)
    res32 = res_state.astype(jnp.float32)
    if tail > 0:
        tail_rows = lax.slice(rows32, (nnz16,), (nnz,))
        tail_cols = lax.slice(cols32, (nnz16,), (nnz,))
        tail_vals = lax.slice(vals32, (nnz16,), (nnz,))
        tail_contrib = tail_vals * jnp.take(res32, tail_cols)
    else:
        tail_rows = jnp.zeros((1,), jnp.int32)
        tail_contrib = jnp.zeros((1,), jnp.float32)
    partials = _sc_partials(rows32, cols32, vals32, res32)
    return _reduce_epilogue(partials, proj_vars.astype(jnp.float32), res32,
                            tail_rows, tail_contrib)


# speedup vs baseline: 1.0353x; 1.0353x over previous
import functools

import jax
import jax.numpy as jnp
from jax import lax
from jax.experimental import pallas as pl
from jax.experimental.pallas import tpu as pltpu
from jax.experimental.pallas import tpu_sc as plsc

_RES = 16384
_LEAK = 0.6
_BIAS = 1.6
_NC = 2
_NS = 16
_NW = _NC * _NS
_CHUNK = 4096
_NBUF = 4
_R0, _R1 = 21, 21


def _sc_partials(rows, cols, vals, res_state):
    nnz = rows.shape[0]
    nnz16 = (nnz // 16) * 16
    n_chunks_total = nnz16 // _CHUNK
    rem = nnz16 - n_chunks_total * _CHUNK
    assert nnz16 >= _CHUNK

    c0_total = (n_chunks_total * _R0) // (_R0 + _R1)
    c1_total = n_chunks_total - c0_total
    q0, r0 = divmod(c0_total, _NS)
    q1, r1 = divmod(c1_total, _NS)
    assert min(q0, q1) >= _NBUF

    mesh = plsc.VectorSubcoreMesh(core_axis_name="c", subcore_axis_name="s")

    @functools.partial(
        pl.kernel,
        out_type=jax.ShapeDtypeStruct((_NW, _RES), jnp.float32),
        mesh=mesh,
        scratch_types=[
            pltpu.VMEM((_RES,), jnp.float32),
            pltpu.VMEM((_RES,), jnp.float32),
            pltpu.VMEM((_NBUF, _CHUNK), jnp.int32),
            pltpu.VMEM((_NBUF, _CHUNK), jnp.int32),
            pltpu.VMEM((_NBUF, _CHUNK), jnp.float32),
            pltpu.VMEM((max(rem, 16),), jnp.int32),
            pltpu.VMEM((max(rem, 16),), jnp.int32),
            pltpu.VMEM((max(rem, 16),), jnp.float32),
            pltpu.SemaphoreType.DMA((_NBUF,)),
            pltpu.SemaphoreType.DMA,
        ],
        compiler_params=pltpu.CompilerParams(needs_layout_passes=False),
    )
    def body(rows_h, cols_h, vals_h, res_h, out_h,
             res_v, acc_v, rbuf, cbuf, vbuf, rrem_v, crem_v, vrem_v,
             sems, res_sem):
        cid = lax.axis_index("c")
        sid = lax.axis_index("s")
        wid = sid * _NC + cid
        is_c0 = cid == 0
        is_mopup = is_c0 & (sid == _NS - 1)
        n_chunks = jnp.where(is_c0, q0 + (sid < r0), q1 + (sid < r1))
        start_chunk = jnp.where(
            is_c0,
            sid * q0 + jnp.minimum(sid, r0),
            c0_total + sid * q1 + jnp.minimum(sid, r1),
        )
        base = start_chunk * _CHUNK

        def issue(ci, slot):
            off = base + ci * _CHUNK
            pltpu.async_copy(rows_h.at[pl.ds(off, _CHUNK)], rbuf.at[slot], sems.at[slot])
            pltpu.async_copy(cols_h.at[pl.ds(off, _CHUNK)], cbuf.at[slot], sems.at[slot])
            pltpu.async_copy(vals_h.at[pl.ds(off, _CHUNK)], vbuf.at[slot], sems.at[slot])

        def drain(slot):
            pltpu.make_async_copy(rows_h.at[pl.ds(0, _CHUNK)], rbuf.at[slot], sems.at[slot]).wait()
            pltpu.make_async_copy(cols_h.at[pl.ds(0, _CHUNK)], cbuf.at[slot], sems.at[slot]).wait()
            pltpu.make_async_copy(vals_h.at[pl.ds(0, _CHUNK)], vbuf.at[slot], sems.at[slot]).wait()

        def gather_scatter(slot, j):
            sl = pl.ds(j * 16, 16)
            g = plsc.load_gather(res_v, [cbuf[slot, sl]])
            plsc.addupdate_scatter(acc_v, [rbuf[slot, sl]], vbuf[slot, sl] * g)

        for b in range(_NBUF):
            issue(b, b)
        res_copy = pltpu.async_copy(res_h, res_v, res_sem)

        zeros = jnp.zeros((16,), jnp.float32)

        @plsc.parallel_loop(0, _RES // 16, unroll=8)
        def _(i):
            acc_v[pl.ds(i * 16, 16)] = zeros

        res_copy.wait()

        def cbody(ci, _):
            slot = lax.rem(ci, _NBUF)
            drain(slot)

            @plsc.parallel_loop(0, _CHUNK // 16, unroll=8)
            def _(j):
                gather_scatter(slot, j)

            @pl.when(ci + _NBUF < n_chunks)
            def _():
                issue(ci + _NBUF, slot)

            return ()

        lax.fori_loop(0, n_chunks, cbody, ())

        if rem > 0:
            roff = pl.multiple_of(jnp.where(sid >= 0, nnz16 - rem, 0), _CHUNK)
            pltpu.async_copy(rows_h.at[pl.ds(roff, rem)], rrem_v, res_sem).wait()
            pltpu.async_copy(cols_h.at[pl.ds(roff, rem)], crem_v, res_sem).wait()
            pltpu.async_copy(vals_h.at[pl.ds(roff, rem)], vrem_v, res_sem).wait()

            @pl.when(is_mopup)
            def _():
                @plsc.parallel_loop(0, rem // 16, unroll=4)
                def _(j):
                    sl = pl.ds(j * 16, 16)
                    g = plsc.load_gather(res_v, [crem_v[sl]])
                    plsc.addupdate_scatter(acc_v, [rrem_v[sl]], vrem_v[sl] * g)

        pltpu.sync_copy(acc_v, out_h.at[wid])

    return body(rows, cols, vals, res_state)


def _reduce_epilogue(partials, proj_vars, res_state, tail_rows, tail_contrib):
    n_tail = tail_rows.shape[0]

    def body(tr_ref, tc_ref, p_ref, pv_ref, rs_ref, o_ref):
        s = jnp.sum(p_ref[...], axis=0)
        iota = lax.broadcasted_iota(jnp.int32, (_RES,), 0)
        for i in range(n_tail):
            s = s + jnp.where(iota == tr_ref[i], tc_ref[i], 0.0)
        act = jnp.tanh(s + pv_ref[...] + _BIAS)
        o_ref[...] = _LEAK * act + (1.0 - _LEAK) * rs_ref[...]

    return pl.pallas_call(
        body,
        out_shape=jax.ShapeDtypeStruct((_RES,), jnp.float32),
        in_specs=[
            pl.BlockSpec(memory_space=pltpu.SMEM),
            pl.BlockSpec(memory_space=pltpu.SMEM),
            pl.BlockSpec(memory_space=pltpu.VMEM),
            pl.BlockSpec(memory_space=pltpu.VMEM),
            pl.BlockSpec(memory_space=pltpu.VMEM),
        ],
    )(tail_rows, tail_contrib, partials, proj_vars, res_state)


def kernel(proj_vars, res_state, rows, cols, vals):
    nnz = rows.shape[0]
    nnz16 = (nnz // 16) * 16
    tail = nnz - nnz16
    rows32 = rows.astype(jnp.int32)
    cols32 = cols.astype(jnp.int32)
    vals32 = vals.astype(jnp.float32)
    res32 = res_state.astype(jnp.float32)
    if tail > 0:
        tail_rows = lax.slice(rows32, (nnz16,), (nnz,))
        tail_cols = lax.slice(cols32, (nnz16,), (nnz,))
        tail_vals = lax.slice(vals32, (nnz16,), (nnz,))
        tail_contrib = tail_vals * jnp.take(res32, tail_cols)
    else:
        tail_rows = jnp.zeros((1,), jnp.int32)
        tail_contrib = jnp.zeros((1,), jnp.float32)
    partials = _sc_partials(rows32, cols32, vals32, res32)
    return _reduce_epilogue(partials, proj_vars.astype(jnp.float32), res32,
                            tail_rows, tail_contrib)
